# baseline (speedup 1.0000x reference)
import functools

import jax
import jax.numpy as jnp
from jax.experimental import pallas as pl
from jax.experimental.pallas import tpu as pltpu

C_PAD = 128
NCH = 4


def _table_kernel(tok_ref, emb_ref, w1_ref, b1_ref, w2p_ref, p_ref, out_ref,
                  w1bf_ref, *, bs, vc):
    i = pl.program_id(0) * 2 + pl.program_id(1)

    @pl.when(pl.program_id(1) == 0)
    def _cast_w1():
        w1bf_ref[...] = w1_ref[...].astype(jnp.bfloat16)

    embc = emb_ref[...].astype(jnp.bfloat16)
    wf = jnp.dot(embc, w1bf_ref[...], preferred_element_type=jnp.float32)
    t = jnp.tanh(wf + b1_ref[...])
    u = jnp.dot(t, w2p_ref[...],
                preferred_element_type=jnp.float32)

    iota = (jax.lax.broadcasted_iota(jnp.int16, (bs, vc), 1)
            + jnp.int16(i * vc))
    oh = (tok_ref[...] == iota).astype(jnp.bfloat16)
    counts = jnp.dot(p_ref[...], oh,
                     preferred_element_type=jnp.float32)

    out_ref[0] = jnp.dot(counts, u,
                         preferred_element_type=jnp.float32)


@jax.jit
def kernel(tokens, emb, w1, b1, w2, b2):
    B, S = tokens.shape
    V, E = emb.shape
    H = w1.shape[1]
    C = w2.shape[1]
    VC = V // NCH
    BS = B * S

    w2p = jnp.zeros((H, C_PAD), jnp.float32).at[:, :C].set(w2) * (1.0 / S)
    row_of = jnp.repeat(jnp.arange(B, dtype=jnp.int32), S)
    p_sel = (jnp.arange(B, dtype=jnp.int32)[:, None] == row_of[None, :]
             ).astype(jnp.bfloat16)
    tok_flat = tokens.reshape(BS, 1).astype(jnp.int16)

    flops = 2 * V * E * H + 2 * B * BS * V + 2 * B * V * C_PAD
    cost = pl.CostEstimate(flops=flops, transcendentals=V * H,
                           bytes_accessed=4 * (V * E + E * H + V * H))

    parts = pl.pallas_call(
        functools.partial(_table_kernel, bs=BS, vc=VC),
        out_shape=jax.ShapeDtypeStruct((NCH, B, C_PAD), jnp.float32),
        grid=(2, NCH // 2),
        in_specs=[
            pl.BlockSpec((BS, 1), lambda i, j: (0, 0)),
            pl.BlockSpec((VC, E), lambda i, j: (i * (NCH // 2) + j, 0)),
            pl.BlockSpec((E, H), lambda i, j: (0, 0)),
            pl.BlockSpec((1, H), lambda i, j: (0, 0)),
            pl.BlockSpec((H, C_PAD), lambda i, j: (0, 0)),
            pl.BlockSpec((B, BS), lambda i, j: (0, 0)),
        ],
        out_specs=pl.BlockSpec((1, B, C_PAD),
                               lambda i, j: (i * (NCH // 2) + j, 0, 0)),
        scratch_shapes=[pltpu.VMEM((E, H), jnp.bfloat16)],
        compiler_params=pltpu.CompilerParams(
            dimension_semantics=("parallel", "arbitrary")),
        cost_estimate=cost,
    )(tok_flat, emb, w1, b1, w2p, p_sel)

    return parts.sum(axis=0)[:, :C] + b2


# speedup vs baseline: 1.0311x; 1.0311x over previous
import functools

import jax
import jax.numpy as jnp
from jax.experimental import pallas as pl
from jax.experimental.pallas import tpu as pltpu

C_PAD = 128
NCH = 4


def _table_kernel(tok_ref, emb_ref, w1_ref, b1_ref, w2p_ref, p_ref, out_ref,
                  *, bs, vc):
    i = pl.program_id(0)

    embc = emb_ref[...].astype(jnp.bfloat16)
    w1c = w1_ref[...].astype(jnp.bfloat16)
    wf = jnp.dot(embc, w1c, preferred_element_type=jnp.float32)
    t = jnp.tanh(wf + b1_ref[...])
    u = jnp.dot(t, w2p_ref[...],
                preferred_element_type=jnp.float32)

    iota = (jax.lax.broadcasted_iota(jnp.int16, (bs, vc), 1)
            + jnp.int16(i * vc))
    oh = (tok_ref[...] == iota).astype(jnp.bfloat16)
    counts = jnp.dot(p_ref[...], oh,
                     preferred_element_type=jnp.float32)

    out_ref[0] = jnp.dot(counts, u,
                         preferred_element_type=jnp.float32)


@jax.jit
def kernel(tokens, emb, w1, b1, w2, b2):
    B, S = tokens.shape
    V, E = emb.shape
    H = w1.shape[1]
    C = w2.shape[1]
    VC = V // NCH
    BS = B * S

    w2p = jnp.zeros((H, C_PAD), jnp.float32).at[:, :C].set(w2) * (1.0 / S)
    row_of = jnp.repeat(jnp.arange(B, dtype=jnp.int32), S)
    p_sel = (jnp.arange(B, dtype=jnp.int32)[:, None] == row_of[None, :]
             ).astype(jnp.bfloat16)
    tok_flat = tokens.reshape(BS, 1).astype(jnp.int16)

    flops = 2 * V * E * H + 2 * B * BS * V + 2 * B * V * C_PAD
    cost = pl.CostEstimate(flops=flops, transcendentals=V * H,
                           bytes_accessed=4 * (V * E + E * H + V * H))

    parts = pl.pallas_call(
        functools.partial(_table_kernel, bs=BS, vc=VC),
        out_shape=jax.ShapeDtypeStruct((NCH, B, C_PAD), jnp.float32),
        grid=(NCH,),
        in_specs=[
            pl.BlockSpec((BS, 1), lambda i: (0, 0)),
            pl.BlockSpec((VC, E), lambda i: (i, 0)),
            pl.BlockSpec((E, H), lambda i: (0, 0)),
            pl.BlockSpec((1, H), lambda i: (0, 0)),
            pl.BlockSpec((H, C_PAD), lambda i: (0, 0)),
            pl.BlockSpec((B, BS), lambda i: (0, 0)),
        ],
        out_specs=pl.BlockSpec((1, B, C_PAD), lambda i: (i, 0, 0)),
        compiler_params=pltpu.CompilerParams(
            dimension_semantics=("parallel",)),
        cost_estimate=cost,
    )(tok_flat, emb, w1, b1, w2p, p_sel)

    return parts.sum(axis=0)[:, :C] + b2


# speedup vs baseline: 1.0358x; 1.0045x over previous
import functools

import jax
import jax.numpy as jnp
from jax.experimental import pallas as pl
from jax.experimental.pallas import tpu as pltpu

C_PAD = 128
NCH = 4


def _table_kernel(tok_ref, emb_ref, w1_ref, b1_ref, w2p_ref, p_ref, out_ref,
                  *, bs, vc):
    i = pl.program_id(0)

    embc = emb_ref[...].astype(jnp.bfloat16)
    w1c = w1_ref[...].astype(jnp.bfloat16)
    wf = jnp.dot(embc, w1c, preferred_element_type=jnp.float32)
    t = jnp.tanh(wf + b1_ref[...])
    u = jnp.dot(t, w2p_ref[...],
                preferred_element_type=jnp.float32)

    iota = (jax.lax.broadcasted_iota(jnp.int16, (bs, vc), 1)
            + jnp.int16(i * vc))
    oh = (tok_ref[...] == iota).astype(jnp.bfloat16)
    counts = jnp.dot(p_ref[...], oh,
                     preferred_element_type=jnp.float32)

    out_ref[0] = jnp.dot(counts, u,
                         preferred_element_type=jnp.float32)


@jax.jit
def kernel(tokens, emb, w1, b1, w2, b2):
    B, S = tokens.shape
    V, E = emb.shape
    H = w1.shape[1]
    C = w2.shape[1]
    VC = V // NCH
    BS = B * S

    w2p = jnp.zeros((H, C_PAD), jnp.float32).at[:, :C].set(w2) * (1.0 / S)
    row_of = jnp.repeat(jnp.arange(B, dtype=jnp.int32), S)
    p_sel = (jnp.arange(B, dtype=jnp.int32)[:, None] == row_of[None, :]
             ).astype(jnp.bfloat16)
    tok_flat = tokens.reshape(BS, 1).astype(jnp.int16)

    flops = 2 * V * E * H + 2 * B * BS * V + 2 * B * V * C_PAD
    cost = pl.CostEstimate(flops=flops, transcendentals=V * H,
                           bytes_accessed=4 * (V * E + E * H + V * H))

    parts = pl.pallas_call(
        functools.partial(_table_kernel, bs=BS, vc=VC),
        out_shape=jax.ShapeDtypeStruct((NCH, B, C_PAD), jnp.float32),
        grid=(NCH,),
        in_specs=[
            pl.BlockSpec((BS, 1), lambda i: (0, 0)),
            pl.BlockSpec((VC, E), lambda i: (i, 0)),
            pl.BlockSpec((E, H), lambda i: (0, 0)),
            pl.BlockSpec((1, H), lambda i: (0, 0)),
            pl.BlockSpec((H, C_PAD), lambda i: (0, 0)),
            pl.BlockSpec((B, BS), lambda i: (0, 0)),
        ],
        out_specs=pl.BlockSpec((1, B, C_PAD), lambda i: (i, 0, 0)),
        compiler_params=pltpu.CompilerParams(
            dimension_semantics=("parallel",)),
    )(tok_flat, emb, w1, b1, w2p, p_sel)

    return parts.sum(axis=0)[:, :C] + b2
